# baseline (speedup 1.0000x reference)
import jax
import jax.numpy as jnp
from jax.experimental import pallas as pl
from jax.experimental.pallas import tpu as pltpu

N = 10000
NP = 10240
K = 5000
KP = 5120
C = 128

PB = 640
XCH = 1024


def _pool_kernel(permcol_ref, tscol_ref, x_ref,
                 gw1_ref, gb1_ref, gw2_ref, gb2_ref, gw3_ref, gb3_ref,
                 pw1_ref, pb1_ref, pw2_ref, pb2_ref, pw3_ref, pb3_ref,
                 gene_ref, pred_ref, xsum_ref):
    pi = pl.program_id(0)
    pv = permcol_ref[...]

    xp = jnp.zeros((PB, C), jnp.float32)
    for c in range(NP // XCH):
        ij = jax.lax.broadcasted_iota(jnp.int32, (1, XCH), 1) + c * XCH
        m = (pv == ij).astype(jnp.float32)
        xp = xp + jax.lax.dot_general(
            m, x_ref[c * XCH:(c + 1) * XCH, :],
            (((1,), (0,)), ((), ())),
            preferred_element_type=jnp.float32,
            precision=jax.lax.Precision.HIGHEST)

    xp = xp * tscol_ref[...]

    h1 = jnp.maximum(jax.lax.dot_general(
        xp, gw1_ref[...], (((1,), (0,)), ((), ())),
        preferred_element_type=jnp.float32) + gb1_ref[...], 0.0)
    h2 = jnp.maximum(jax.lax.dot_general(
        h1, gw2_ref[...], (((1,), (0,)), ((), ())),
        preferred_element_type=jnp.float32) + gb2_ref[...], 0.0)
    gene_ref[...] = jax.lax.dot_general(
        h2, gw3_ref[...], (((1,), (0,)), ((), ())),
        preferred_element_type=jnp.float32) + gb3_ref[...]

    @pl.when(pi == 0)
    def _init():
        xsum_ref[...] = jnp.zeros((8, C), jnp.float32)

    pcol = jax.lax.broadcasted_iota(jnp.int32, (PB, 1), 0) + pi * PB
    valid = (pcol < K).astype(jnp.float32)
    xsum_ref[0:1, :] = xsum_ref[0:1, :] + jnp.sum(
        xp * valid, axis=0, keepdims=True)

    @pl.when(pi == KP // PB - 1)
    def _pred():
        gr = xsum_ref[0:1, :] / jnp.float32(K)
        gr8 = jnp.broadcast_to(gr, (8, C))
        t1 = jnp.maximum(jax.lax.dot_general(
            gr8, pw1_ref[...], (((1,), (0,)), ((), ())),
            preferred_element_type=jnp.float32) + pb1_ref[...], 0.0)
        t2 = jnp.maximum(jax.lax.dot_general(
            t1, pw2_ref[...], (((1,), (0,)), ((), ())),
            preferred_element_type=jnp.float32) + pb2_ref[...], 0.0)
        pred_ref[...] = jax.lax.dot_general(
            t2, pw3_ref[...], (((1,), (0,)), ((), ())),
            preferred_element_type=jnp.float32) + pb3_ref[...]


def kernel(x, edge_index, batch, gnn_w_rel, gnn_b_rel, gnn_w_root,
           pred_w1, pred_b1, pred_w2, pred_b2, pred_w3, pred_b3,
           gene_w1, gene_b1, gene_w2, gene_b2, gene_w3, gene_b3):
    src = edge_index[0]
    dst = edge_index[1]
    agg = jax.ops.segment_sum(x[src], dst, num_segments=N)
    score = jnp.tanh((agg @ gnn_w_rel + gnn_b_rel) + (x @ gnn_w_root)).reshape(-1)
    top_scores, perm = jax.lax.top_k(score, K)

    permcol = jnp.concatenate(
        [perm, jnp.full((KP - K,), NP - 1, jnp.int32)]).reshape(KP, 1)
    tscol = jnp.concatenate(
        [top_scores, jnp.zeros((KP - K,), jnp.float32)]).reshape(KP, 1)
    xpad = jnp.concatenate([x, jnp.zeros((NP - N, C), jnp.float32)], axis=0)

    gw3p = jnp.zeros((C, 128), jnp.float32).at[:, 0:1].set(gene_w3)
    gb3p = jnp.zeros((1, 128), jnp.float32).at[0, 0].set(gene_b3[0])
    pw3p = jnp.zeros((C, 128), jnp.float32).at[:, 0:10].set(pred_w3)
    pb3p = jnp.zeros((1, 128), jnp.float32).at[0, 0:10].set(pred_b3)

    full = lambda shape: pl.BlockSpec(shape, lambda i: (0, 0))
    gene_p, pred_p = pl.pallas_call(
        _pool_kernel,
        grid=(KP // PB,),
        in_specs=[
            pl.BlockSpec((PB, 1), lambda i: (i, 0)),
            pl.BlockSpec((PB, 1), lambda i: (i, 0)),
            full((NP, C)),
            full((C, 256)), full((1, 256)),
            full((256, C)), full((1, C)),
            full((C, 128)), full((1, 128)),
            full((C, 256)), full((1, 256)),
            full((256, C)), full((1, C)),
            full((C, 128)), full((1, 128)),
        ],
        out_specs=[
            pl.BlockSpec((PB, 128), lambda i: (i, 0)),
            pl.BlockSpec((8, 128), lambda i: (0, 0)),
        ],
        out_shape=[
            jax.ShapeDtypeStruct((KP, 128), jnp.float32),
            jax.ShapeDtypeStruct((8, 128), jnp.float32),
        ],
        scratch_shapes=[pltpu.VMEM((8, C), jnp.float32)],
    )(permcol, tscol, xpad,
      gene_w1, gene_b1.reshape(1, 256), gene_w2, gene_b2.reshape(1, C),
      gw3p, gb3p,
      pred_w1, pred_b1.reshape(1, 256), pred_w2, pred_b2.reshape(1, C),
      pw3p, pb3p)

    pred = pred_p[0:1, 0:10]
    gene_scores = gene_p[:K, 0:1]
    return (pred, gene_scores, perm)


# speedup vs baseline: 1.0875x; 1.0875x over previous
import jax
import jax.numpy as jnp
from jax.experimental import pallas as pl
from jax.experimental.pallas import tpu as pltpu

N = 10000
NP = 10240
K = 5000
KP = 5120
C = 128

PB = 640
XCH = 1024


def _pool_kernel(xg_ref, tscol_ref,
                 gw1_ref, gb1_ref, gw2_ref, gb2_ref, gw3_ref, gb3_ref,
                 pw1_ref, pb1_ref, pw2_ref, pb2_ref, pw3_ref, pb3_ref,
                 gene_ref, pred_ref, xsum_ref):
    pi = pl.program_id(0)
    xp = xg_ref[...] * tscol_ref[...]

    h1 = jnp.maximum(jax.lax.dot_general(
        xp, gw1_ref[...], (((1,), (0,)), ((), ())),
        preferred_element_type=jnp.float32) + gb1_ref[...], 0.0)
    h2 = jnp.maximum(jax.lax.dot_general(
        h1, gw2_ref[...], (((1,), (0,)), ((), ())),
        preferred_element_type=jnp.float32) + gb2_ref[...], 0.0)
    gene_ref[...] = jax.lax.dot_general(
        h2, gw3_ref[...], (((1,), (0,)), ((), ())),
        preferred_element_type=jnp.float32) + gb3_ref[...]

    @pl.when(pi == 0)
    def _init():
        xsum_ref[...] = jnp.zeros((8, C), jnp.float32)

    pcol = jax.lax.broadcasted_iota(jnp.int32, (PB, 1), 0) + pi * PB
    valid = (pcol < K).astype(jnp.float32)
    xsum_ref[0:1, :] = xsum_ref[0:1, :] + jnp.sum(
        xp * valid, axis=0, keepdims=True)

    @pl.when(pi == KP // PB - 1)
    def _pred():
        gr = xsum_ref[0:1, :] / jnp.float32(K)
        gr8 = jnp.broadcast_to(gr, (8, C))
        t1 = jnp.maximum(jax.lax.dot_general(
            gr8, pw1_ref[...], (((1,), (0,)), ((), ())),
            preferred_element_type=jnp.float32) + pb1_ref[...], 0.0)
        t2 = jnp.maximum(jax.lax.dot_general(
            t1, pw2_ref[...], (((1,), (0,)), ((), ())),
            preferred_element_type=jnp.float32) + pb2_ref[...], 0.0)
        pred_ref[...] = jax.lax.dot_general(
            t2, pw3_ref[...], (((1,), (0,)), ((), ())),
            preferred_element_type=jnp.float32) + pb3_ref[...]


def kernel(x, edge_index, batch, gnn_w_rel, gnn_b_rel, gnn_w_root,
           pred_w1, pred_b1, pred_w2, pred_b2, pred_w3, pred_b3,
           gene_w1, gene_b1, gene_w2, gene_b2, gene_w3, gene_b3):
    src = edge_index[0]
    dst = edge_index[1]
    agg = jax.ops.segment_sum(x[src], dst, num_segments=N)
    score = jnp.tanh((agg @ gnn_w_rel + gnn_b_rel) + (x @ gnn_w_root)).reshape(-1)
    top_scores, perm = jax.lax.top_k(score, K)

    tscol = jnp.concatenate(
        [top_scores, jnp.zeros((KP - K,), jnp.float32)]).reshape(KP, 1)
    xg = jnp.concatenate(
        [x[perm], jnp.zeros((KP - K, C), jnp.float32)], axis=0)

    gw3p = jnp.zeros((C, 128), jnp.float32).at[:, 0:1].set(gene_w3)
    gb3p = jnp.zeros((1, 128), jnp.float32).at[0, 0].set(gene_b3[0])
    pw3p = jnp.zeros((C, 128), jnp.float32).at[:, 0:10].set(pred_w3)
    pb3p = jnp.zeros((1, 128), jnp.float32).at[0, 0:10].set(pred_b3)

    full = lambda shape: pl.BlockSpec(shape, lambda i: (0, 0))
    gene_p, pred_p = pl.pallas_call(
        _pool_kernel,
        grid=(KP // PB,),
        in_specs=[
            pl.BlockSpec((PB, C), lambda i: (i, 0)),
            pl.BlockSpec((PB, 1), lambda i: (i, 0)),
            full((C, 256)), full((1, 256)),
            full((256, C)), full((1, C)),
            full((C, 128)), full((1, 128)),
            full((C, 256)), full((1, 256)),
            full((256, C)), full((1, C)),
            full((C, 128)), full((1, 128)),
        ],
        out_specs=[
            pl.BlockSpec((PB, 128), lambda i: (i, 0)),
            pl.BlockSpec((8, 128), lambda i: (0, 0)),
        ],
        out_shape=[
            jax.ShapeDtypeStruct((KP, 128), jnp.float32),
            jax.ShapeDtypeStruct((8, 128), jnp.float32),
        ],
        scratch_shapes=[pltpu.VMEM((8, C), jnp.float32)],
    )(xg, tscol,
      gene_w1, gene_b1.reshape(1, 256), gene_w2, gene_b2.reshape(1, C),
      gw3p, gb3p,
      pred_w1, pred_b1.reshape(1, 256), pred_w2, pred_b2.reshape(1, C),
      pw3p, pb3p)

    pred = pred_p[0:1, 0:10]
    gene_scores = gene_p[:K, 0:1]
    return (pred, gene_scores, perm)


# speedup vs baseline: 1.0901x; 1.0024x over previous
import jax
import jax.numpy as jnp
from jax.experimental import pallas as pl

N = 10000
NP = 10240
K = 5000
KP = 5120
C = 128


def _pool_kernel(xg_ref, tscol_ref,
                 gw1_ref, gb1_ref, gw2_ref, gb2_ref, gw3_ref, gb3_ref,
                 pw1_ref, pb1_ref, pw2_ref, pb2_ref, pw3_ref, pb3_ref,
                 gene_ref, pred_ref):
    xp = xg_ref[...] * tscol_ref[...]

    h1 = jnp.maximum(jax.lax.dot_general(
        xp, gw1_ref[...], (((1,), (0,)), ((), ())),
        preferred_element_type=jnp.float32) + gb1_ref[...], 0.0)
    h2 = jnp.maximum(jax.lax.dot_general(
        h1, gw2_ref[...], (((1,), (0,)), ((), ())),
        preferred_element_type=jnp.float32) + gb2_ref[...], 0.0)
    gene_ref[...] = jax.lax.dot_general(
        h2, gw3_ref[...], (((1,), (0,)), ((), ())),
        preferred_element_type=jnp.float32) + gb3_ref[...]

    pcol = jax.lax.broadcasted_iota(jnp.int32, (KP, 1), 0)
    valid = (pcol < K).astype(jnp.float32)
    gr = jnp.sum(xp * valid, axis=0, keepdims=True) / jnp.float32(K)
    gr8 = jnp.broadcast_to(gr, (8, C))
    t1 = jnp.maximum(jax.lax.dot_general(
        gr8, pw1_ref[...], (((1,), (0,)), ((), ())),
        preferred_element_type=jnp.float32) + pb1_ref[...], 0.0)
    t2 = jnp.maximum(jax.lax.dot_general(
        t1, pw2_ref[...], (((1,), (0,)), ((), ())),
        preferred_element_type=jnp.float32) + pb2_ref[...], 0.0)
    pred_ref[...] = jax.lax.dot_general(
        t2, pw3_ref[...], (((1,), (0,)), ((), ())),
        preferred_element_type=jnp.float32) + pb3_ref[...]


def kernel(x, edge_index, batch, gnn_w_rel, gnn_b_rel, gnn_w_root,
           pred_w1, pred_b1, pred_w2, pred_b2, pred_w3, pred_b3,
           gene_w1, gene_b1, gene_w2, gene_b2, gene_w3, gene_b3):
    src = edge_index[0]
    dst = edge_index[1]
    agg = jax.ops.segment_sum(x[src], dst, num_segments=N)
    score = jnp.tanh((agg @ gnn_w_rel + gnn_b_rel) + (x @ gnn_w_root)).reshape(-1)
    top_scores, perm = jax.lax.top_k(score, K)

    tscol = jnp.concatenate(
        [top_scores, jnp.zeros((KP - K,), jnp.float32)]).reshape(KP, 1)
    xg = jnp.concatenate(
        [x[perm], jnp.zeros((KP - K, C), jnp.float32)], axis=0)

    gw3p = jnp.zeros((C, 128), jnp.float32).at[:, 0:1].set(gene_w3)
    gb3p = jnp.zeros((1, 128), jnp.float32).at[0, 0].set(gene_b3[0])
    pw3p = jnp.zeros((C, 128), jnp.float32).at[:, 0:10].set(pred_w3)
    pb3p = jnp.zeros((1, 128), jnp.float32).at[0, 0:10].set(pred_b3)

    gene_p, pred_p = pl.pallas_call(
        _pool_kernel,
        out_shape=[
            jax.ShapeDtypeStruct((KP, 128), jnp.float32),
            jax.ShapeDtypeStruct((8, 128), jnp.float32),
        ],
    )(xg, tscol,
      gene_w1, gene_b1.reshape(1, 256), gene_w2, gene_b2.reshape(1, C),
      gw3p, gb3p,
      pred_w1, pred_b1.reshape(1, 256), pred_w2, pred_b2.reshape(1, C),
      pw3p, pb3p)

    pred = pred_p[0:1, 0:10]
    gene_scores = gene_p[:K, 0:1]
    return (pred, gene_scores, perm)
